# Initial kernel scaffold; baseline (speedup 1.0000x reference)
import jax
import jax.numpy as jnp
from jax.experimental import pallas as pl


def kernel(words, suffix, prefix, emb_word, emb_pref, emb_suff, W1, b1, W2, b2):
    raise NotImplementedError("write your pallas kernel here")


# speedup vs baseline: 1.7844x; 1.7844x over previous
import jax
import jax.numpy as jnp
from jax import lax
from jax.experimental import pallas as pl
from jax.experimental.pallas import tpu as pltpu
from jax.experimental.pallas import tpu_sc as plsc

B = 4096
WL = 5
D = 64
H = 128
T = 50
NI = B * WL

NC, NS = 2, 16
NW = NC * NS
PER_W = NI // NW
CHUNK = 128
NCHUNK = PER_W // CHUNK

BB = 512


def _sc_gather_body(ew, ep, es, wi, pi, si, ow, op_, os_,
                    wv, pv, sv, rw, rp, rs, sem):
    wid = lax.axis_index("s") * NC + lax.axis_index("c")
    base = wid * PER_W
    pltpu.sync_copy(wi.at[pl.ds(base, PER_W)], wv)
    pltpu.sync_copy(pi.at[pl.ds(base, PER_W)], pv)
    pltpu.sync_copy(si.at[pl.ds(base, PER_W)], sv)
    for c in range(NCHUNK):
        sl = pl.ds(c * CHUNK, CHUNK)
        pltpu.async_copy(ew.at[wv.at[sl]], rw.at[sl], sem).wait()
        pltpu.async_copy(ep.at[pv.at[sl]], rp.at[sl], sem).wait()
        pltpu.async_copy(es.at[sv.at[sl]], rs.at[sl], sem).wait()
    pltpu.sync_copy(rw, ow.at[pl.ds(base, PER_W)])
    pltpu.sync_copy(rp, op_.at[pl.ds(base, PER_W)])
    pltpu.sync_copy(rs, os_.at[pl.ds(base, PER_W)])


def _sc_gather(emb_word, emb_pref, emb_suff, widx, pidx, sidx):
    mesh = plsc.VectorSubcoreMesh(core_axis_name="c", subcore_axis_name="s")
    out_t = [jax.ShapeDtypeStruct((NI, D), jnp.float32)] * 3
    scratch = [
        pltpu.VMEM((PER_W,), jnp.int32),
        pltpu.VMEM((PER_W,), jnp.int32),
        pltpu.VMEM((PER_W,), jnp.int32),
        pltpu.VMEM((PER_W, D), jnp.float32),
        pltpu.VMEM((PER_W, D), jnp.float32),
        pltpu.VMEM((PER_W, D), jnp.float32),
        pltpu.SemaphoreType.DMA,
    ]
    k = pl.kernel(_sc_gather_body, out_type=out_t, mesh=mesh,
                  scratch_types=scratch,
                  compiler_params=pltpu.CompilerParams(
                      use_tc_tiling_on_sc=False))
    return k(emb_word, emb_pref, emb_suff, widx, pidx, sidx)


def _mlp_body(hw, hp, hs, w1, b1, w2, b2, out):
    avg = (hw[...] + hp[...] + hs[...]) * (1.0 / 3.0)
    h2 = jnp.tanh(
        jnp.dot(avg, w1[...], preferred_element_type=jnp.float32,
                precision=lax.Precision.HIGHEST) + b1[...])
    o = jnp.dot(h2, w2[...], preferred_element_type=jnp.float32,
                precision=lax.Precision.HIGHEST) + b2[...]
    m = jnp.max(o, axis=1, keepdims=True)
    s = o - m
    lse = jnp.log(jnp.sum(jnp.exp(s), axis=1, keepdims=True))
    out[...] = s - lse


def _mlp(hw, hp, hs, W1, b1, W2, b2, *, interpret=False):
    x_spec = pl.BlockSpec((BB, WL * D), lambda i: (i, 0))
    return pl.pallas_call(
        _mlp_body,
        grid=(B // BB,),
        in_specs=[
            x_spec, x_spec, x_spec,
            pl.BlockSpec((WL * D, H), lambda i: (0, 0)),
            pl.BlockSpec((1, H), lambda i: (0, 0)),
            pl.BlockSpec((H, T), lambda i: (0, 0)),
            pl.BlockSpec((1, T), lambda i: (0, 0)),
        ],
        out_specs=pl.BlockSpec((BB, T), lambda i: (i, 0)),
        out_shape=jax.ShapeDtypeStruct((B, T), jnp.float32),
        interpret=interpret,
    )(hw, hp, hs, W1, b1.reshape(1, H), W2, b2.reshape(1, T))


def kernel(words, suffix, prefix, emb_word, emb_pref, emb_suff, W1, b1, W2, b2):
    widx = words.reshape(NI)
    pidx = prefix.reshape(NI)
    sidx = suffix.reshape(NI)
    hw, hp, hs = _sc_gather(emb_word, emb_pref, emb_suff, widx, pidx, sidx)
    hw = hw.reshape(B, WL * D)
    hp = hp.reshape(B, WL * D)
    hs = hs.reshape(B, WL * D)
    return _mlp(hw, hp, hs, W1, b1, W2, b2)


# speedup vs baseline: 1.8842x; 1.0559x over previous
import jax
import jax.numpy as jnp
from jax import lax
from jax.experimental import pallas as pl
from jax.experimental.pallas import tpu as pltpu
from jax.experimental.pallas import tpu_sc as plsc

B = 4096
WL = 5
D = 64
H = 128
T = 50
NI = B * WL

NC, NS = 2, 16
NW = NC * NS
PER_W = NI // NW
CHUNK = 128
NCHUNK = PER_W // CHUNK

BB = 512


def _sc_gather_body(ew, ep, es, wi, pi, si, ow, op_, os_,
                    wv, pv, sv, rw, rp, rs, sem):
    wid = lax.axis_index("s") * NC + lax.axis_index("c")
    base = wid * PER_W
    idx_cps = [
        pltpu.async_copy(wi.at[pl.ds(base, PER_W)], wv, sem),
        pltpu.async_copy(pi.at[pl.ds(base, PER_W)], pv, sem),
        pltpu.async_copy(si.at[pl.ds(base, PER_W)], sv, sem),
    ]
    for cp in idx_cps:
        cp.wait()
    gather_cps = []
    for c in range(NCHUNK):
        sl = pl.ds(c * CHUNK, CHUNK)
        gather_cps.append(pltpu.async_copy(ew.at[wv.at[sl]], rw.at[sl], sem))
        gather_cps.append(pltpu.async_copy(ep.at[pv.at[sl]], rp.at[sl], sem))
        gather_cps.append(pltpu.async_copy(es.at[sv.at[sl]], rs.at[sl], sem))
    for cp in gather_cps:
        cp.wait()
    out_cps = [
        pltpu.async_copy(rw, ow.at[pl.ds(base, PER_W)], sem),
        pltpu.async_copy(rp, op_.at[pl.ds(base, PER_W)], sem),
        pltpu.async_copy(rs, os_.at[pl.ds(base, PER_W)], sem),
    ]
    for cp in out_cps:
        cp.wait()


def _sc_gather(emb_word, emb_pref, emb_suff, widx, pidx, sidx):
    mesh = plsc.VectorSubcoreMesh(core_axis_name="c", subcore_axis_name="s")
    out_t = [jax.ShapeDtypeStruct((NI, D), jnp.float32)] * 3
    scratch = [
        pltpu.VMEM((PER_W,), jnp.int32),
        pltpu.VMEM((PER_W,), jnp.int32),
        pltpu.VMEM((PER_W,), jnp.int32),
        pltpu.VMEM((PER_W, D), jnp.float32),
        pltpu.VMEM((PER_W, D), jnp.float32),
        pltpu.VMEM((PER_W, D), jnp.float32),
        pltpu.SemaphoreType.DMA,
    ]
    k = pl.kernel(_sc_gather_body, out_type=out_t, mesh=mesh,
                  scratch_types=scratch,
                  compiler_params=pltpu.CompilerParams(
                      use_tc_tiling_on_sc=False))
    return k(emb_word, emb_pref, emb_suff, widx, pidx, sidx)


def _mlp_body(hw, hp, hs, w1, b1, w2, b2, out):
    avg = (hw[...] + hp[...] + hs[...]) * (1.0 / 3.0)
    h2 = jnp.tanh(
        jnp.dot(avg, w1[...], preferred_element_type=jnp.float32,
                precision=lax.Precision.HIGHEST) + b1[...])
    o = jnp.dot(h2, w2[...], preferred_element_type=jnp.float32,
                precision=lax.Precision.HIGHEST) + b2[...]
    m = jnp.max(o, axis=1, keepdims=True)
    s = o - m
    lse = jnp.log(jnp.sum(jnp.exp(s), axis=1, keepdims=True))
    out[...] = s - lse


def _mlp(hw, hp, hs, W1, b1, W2, b2, *, interpret=False):
    x_spec = pl.BlockSpec((BB, WL * D), lambda i: (i, 0))
    return pl.pallas_call(
        _mlp_body,
        grid=(B // BB,),
        in_specs=[
            x_spec, x_spec, x_spec,
            pl.BlockSpec((WL * D, H), lambda i: (0, 0)),
            pl.BlockSpec((1, H), lambda i: (0, 0)),
            pl.BlockSpec((H, T), lambda i: (0, 0)),
            pl.BlockSpec((1, T), lambda i: (0, 0)),
        ],
        out_specs=pl.BlockSpec((BB, T), lambda i: (i, 0)),
        out_shape=jax.ShapeDtypeStruct((B, T), jnp.float32),
        interpret=interpret,
    )(hw, hp, hs, W1, b1.reshape(1, H), W2, b2.reshape(1, T))


def kernel(words, suffix, prefix, emb_word, emb_pref, emb_suff, W1, b1, W2, b2):
    widx = words.reshape(NI)
    pidx = prefix.reshape(NI)
    sidx = suffix.reshape(NI)
    hw, hp, hs = _sc_gather(emb_word, emb_pref, emb_suff, widx, pidx, sidx)
    hw = hw.reshape(B, WL * D)
    hp = hp.reshape(B, WL * D)
    hs = hs.reshape(B, WL * D)
    return _mlp(hw, hp, hs, W1, b1, W2, b2)
